# baseline (speedup 1.0000x reference)
import functools

import jax
import jax.numpy as jnp
from jax.experimental import pallas as pl
from jax.experimental.pallas import tpu as pltpu

_TOPK = 8


def _route(score):
    num_e = score.shape[-1]
    col = jax.lax.broadcasted_iota(jnp.int32, score.shape, 1)
    neg = jnp.float32(-jnp.inf)
    work = score
    m0 = None
    idxs = []
    for j in range(_TOPK):
        mx = jnp.max(work, axis=1, keepdims=True)
        if j == 0:
            m0 = mx
        amx = jnp.min(jnp.where(work == mx, col, num_e), axis=1,
                      keepdims=True)
        idxs.append(amx)
        work = jnp.where(col == amx, neg, work)
    sel = work == neg
    p = jnp.where(sel, jnp.exp(score - m0), 0.0)
    return p / jnp.sum(p, axis=1, keepdims=True), jnp.concatenate(idxs, axis=1)


def _body(x_ref, w1_ref, b1_ref, w2_ref, b2_ref, router_ref, idx_ref, h_acc,
          *, nsteps_k):
    k = pl.program_id(1)

    @pl.when(k == 0)
    def _():
        h_acc[...] = jnp.zeros_like(h_acc)

    h_acc[...] += jnp.dot(x_ref[...], w1_ref[...],
                          preferred_element_type=jnp.float32)

    @pl.when(k == nsteps_k - 1)
    def _():
        h = jnp.maximum(h_acc[...] + b1_ref[...], 0.0)
        score = jnp.dot(h, w2_ref[...],
                        preferred_element_type=jnp.float32) + b2_ref[...]
        router, idx = _route(score)
        router_ref[...] = router
        idx_ref[...] = idx


def kernel(inputs, W1, b1, W2, b2):
    m, k_dim = inputs.shape
    n_dim = W1.shape[1]
    num_e = W2.shape[1]
    bm, bk = min(2048, m), min(128, k_dim)
    grid = (m // bm, k_dim // bk)

    router, idx = pl.pallas_call(
        functools.partial(_body, nsteps_k=grid[1]),
        grid=grid,
        in_specs=[
            pl.BlockSpec((bm, bk), lambda i, k: (i, k)),
            pl.BlockSpec((bk, n_dim), lambda i, k: (k, 0)),
            pl.BlockSpec((1, n_dim), lambda i, k: (0, 0)),
            pl.BlockSpec((n_dim, num_e), lambda i, k: (0, 0)),
            pl.BlockSpec((1, num_e), lambda i, k: (0, 0)),
        ],
        out_specs=[
            pl.BlockSpec((bm, num_e), lambda i, k: (i, 0)),
            pl.BlockSpec((bm, _TOPK), lambda i, k: (i, 0)),
        ],
        out_shape=[
            jax.ShapeDtypeStruct((m, num_e), jnp.float32),
            jax.ShapeDtypeStruct((m, _TOPK), jnp.int32),
        ],
        scratch_shapes=[pltpu.VMEM((bm, n_dim), jnp.float32)],
        compiler_params=pltpu.CompilerParams(
            dimension_semantics=("parallel", "arbitrary")),
    )(inputs, W1, b1.reshape(1, n_dim), W2, b2.reshape(1, num_e))
    return router, idx


# speedup vs baseline: 1.7839x; 1.7839x over previous
import functools

import jax
import jax.numpy as jnp
from jax.experimental import pallas as pl
from jax.experimental.pallas import tpu as pltpu

_TOPK = 8


def _route(score):
    num_e = score.shape[-1]
    col = jax.lax.broadcasted_iota(jnp.int32, score.shape, 1)
    neg = jnp.float32(-jnp.inf)
    work = score
    m0 = None
    idxs = []
    for j in range(_TOPK):
        mx = jnp.max(work, axis=1, keepdims=True)
        if j == 0:
            m0 = mx
        amx = jnp.min(jnp.where(work == mx, col, num_e), axis=1,
                      keepdims=True)
        idxs.append(amx)
        work = jnp.where(col == amx, neg, work)
    sel = work == neg
    p = jnp.where(sel, jnp.exp(score - m0), 0.0)
    return p / jnp.sum(p, axis=1, keepdims=True), jnp.concatenate(idxs, axis=1)


def _body(x_ref, w1_ref, b1_ref, w2_ref, b2_ref, router_ref, idx_ref, h_acc,
          *, nsteps_k):
    k = pl.program_id(1)

    @pl.when(k == 0)
    def _():
        h_acc[...] = jnp.zeros_like(h_acc)

    h_acc[...] += jnp.dot(x_ref[...], w1_ref[...],
                          preferred_element_type=jnp.float32)

    @pl.when(k == nsteps_k - 1)
    def _():
        h = jnp.maximum(h_acc[...] + b1_ref[...], 0.0)
        score = jnp.dot(h, w2_ref[...],
                        preferred_element_type=jnp.float32) + b2_ref[...]
        router, idx = _route(score)
        router_ref[...] = router
        idx_ref[...] = idx


def kernel(inputs, W1, b1, W2, b2):
    m, k_dim = inputs.shape
    n_dim = W1.shape[1]
    num_e = W2.shape[1]
    bm, bk = min(2048, m), min(256, k_dim)
    grid = (m // bm, k_dim // bk)

    router, idx = pl.pallas_call(
        functools.partial(_body, nsteps_k=grid[1]),
        grid=grid,
        in_specs=[
            pl.BlockSpec((bm, bk), lambda i, k: (i, k)),
            pl.BlockSpec((bk, n_dim), lambda i, k: (k, 0)),
            pl.BlockSpec((1, n_dim), lambda i, k: (0, 0)),
            pl.BlockSpec((n_dim, num_e), lambda i, k: (0, 0)),
            pl.BlockSpec((1, num_e), lambda i, k: (0, 0)),
        ],
        out_specs=[
            pl.BlockSpec((bm, num_e), lambda i, k: (i, 0)),
            pl.BlockSpec((bm, _TOPK), lambda i, k: (i, 0)),
        ],
        out_shape=[
            jax.ShapeDtypeStruct((m, num_e), jnp.float32),
            jax.ShapeDtypeStruct((m, _TOPK), jnp.int32),
        ],
        scratch_shapes=[pltpu.VMEM((bm, n_dim), jnp.float32)],
        compiler_params=pltpu.CompilerParams(
            dimension_semantics=("parallel", "arbitrary"),
            vmem_limit_bytes=100 * 1024 * 1024),
    )(inputs, W1, b1.reshape(1, n_dim), W2, b2.reshape(1, num_e))
    return router, idx
